# baseline (speedup 1.0000x reference)
import functools

import jax
import jax.numpy as jnp
from jax import lax
from jax.experimental import pallas as pl
from jax.experimental.pallas import tpu as pltpu
from jax.experimental.pallas import tpu_sc as plsc

_TOTAL_HEADS = 32
_NC, _NS = 2, 16


def _tc_body(li_ref, cp_ref, xk_ref, xv_ref, kc_ref, vc_ref, ko_ref, vo_ref):
    bs = ko_ref.shape[2]
    insert = xk_ref.shape[1]
    li = li_ref[0]
    cp = cp_ref[0]
    start = pl.program_id(1) * bs

    ko_ref[...] = kc_ref[...]
    vo_ref[...] = vc_ref[...]
    for i in range(insert):
        lr = cp + i - start
        @pl.when((lr >= 0) & (lr < bs))
        def _():
            ko_ref[li, 0, lr] = xk_ref[0, i]
            vo_ref[li, 0, lr] = xv_ref[0, i]


def _cache_update_tc(xk, xv, k_cache, v_cache, li, cp):
    L, B, S, H, D = k_cache.shape
    insert = xk.shape[1]
    bs = 512
    grid = (B, S // bs)
    cache_spec = pl.BlockSpec((L, 1, bs, H, D), lambda b, s: (0, b, s, 0, 0))
    x_spec = pl.BlockSpec((1, insert, H, D), lambda b, s: (b, 0, 0, 0))
    return pl.pallas_call(
        _tc_body,
        grid=grid,
        in_specs=[
            pl.BlockSpec(memory_space=pltpu.SMEM),
            pl.BlockSpec(memory_space=pltpu.SMEM),
            x_spec, x_spec, cache_spec, cache_spec,
        ],
        out_specs=[cache_spec, cache_spec],
        out_shape=[
            jax.ShapeDtypeStruct(k_cache.shape, k_cache.dtype),
            jax.ShapeDtypeStruct(v_cache.shape, v_cache.dtype),
        ],
        compiler_params=pltpu.CompilerParams(
            dimension_semantics=("parallel", "parallel"),
        ),
    )(li.reshape(1), cp.reshape(1), xk, xv, k_cache, v_cache)


def _repeat_sc(xk, xv, k_cache, v_cache, licp):
    L, B, S, H, D = k_cache.shape
    insert = xk.shape[1]
    rep = _TOTAL_HEADS // H
    nw = _NC * _NS
    qs = (B * S) // nw
    nq = S // qs
    ch = 128

    kc2 = k_cache.reshape(L, B, S, H * D)
    vc2 = v_cache.reshape(L, B, S, H * D)
    xk2 = xk.reshape(B, insert, H * D)
    xv2 = xv.reshape(B, insert, H * D)

    mesh = plsc.VectorSubcoreMesh(core_axis_name="c", subcore_axis_name="s")

    @functools.partial(
        pl.kernel,
        out_type=[
            jax.ShapeDtypeStruct((B, S, _TOTAL_HEADS * D), xk.dtype),
            jax.ShapeDtypeStruct((B, S, _TOTAL_HEADS * D), xv.dtype),
        ],
        mesh=mesh,
        scratch_types=[
            pltpu.VMEM((ch, H * D), k_cache.dtype),
            pltpu.VMEM((insert, H * D), xk.dtype),
            pltpu.VMEM((16,), jnp.int32),
            pltpu.SemaphoreType.DMA,
        ],
    )
    def sc_kernel(licp_hbm, xk_hbm, xv_hbm, kc_hbm, vc_hbm,
                  keys_hbm, vals_hbm, chunk_v, xbuf_v, licp_v, sem):
        wid = lax.axis_index("s") * _NC + lax.axis_index("c")
        b = wid // nq
        q = wid % nq
        pltpu.sync_copy(licp_hbm, licp_v)
        licp_vec = licp_v[...]
        li = licp_vec[0]
        cp = pl.multiple_of(licp_vec[1], 8)

        def one_tensor(src_hbm, x_hbm, dst_hbm):
            def chunk_body(i, carry):
                s0 = q * qs + i * ch
                pltpu.sync_copy(src_hbm.at[li, b, pl.ds(s0, ch)], chunk_v)
                copies = []
                for h in range(H):
                    for t in range(rep):
                        copies.append(pltpu.async_copy(
                            chunk_v.at[:, pl.ds(h * D, D)],
                            dst_hbm.at[b, pl.ds(s0, ch),
                                       pl.ds((h * rep + t) * D, D)],
                            sem))
                for c_ in copies:
                    c_.wait()
                return carry
            lax.fori_loop(0, qs // ch, chunk_body, 0)
            pltpu.sync_copy(x_hbm.at[b], xbuf_v)
            @pl.when((cp >= q * qs) & (cp < (q + 1) * qs))
            def _():
                patches = []
                for h in range(H):
                    for t in range(rep):
                        patches.append(pltpu.async_copy(
                            xbuf_v.at[:, pl.ds(h * D, D)],
                            dst_hbm.at[b, pl.ds(cp, insert),
                                       pl.ds((h * rep + t) * D, D)],
                            sem))
                for p_ in patches:
                    p_.wait()

        one_tensor(kc_hbm, xk_hbm, keys_hbm)
        one_tensor(vc_hbm, xv_hbm, vals_hbm)

    keys, values = sc_kernel(licp, xk2, xv2, kc2, vc2)
    return (keys.reshape(B, S, _TOTAL_HEADS, D),
            values.reshape(B, S, _TOTAL_HEADS, D))


def kernel(xk, xv, k_cache, v_cache, layer_idx, cur_pos, n_rep):
    L, B, S, H, D = k_cache.shape
    insert = xk.shape[1]
    li = jnp.clip(jnp.asarray(layer_idx, jnp.int32), 0, L - 1)
    cp = jnp.clip(jnp.asarray(cur_pos, jnp.int32), 0, S - insert)
    licp = jnp.zeros((16,), jnp.int32).at[0].set(li).at[1].set(cp)
    keys, values = _repeat_sc(xk, xv, k_cache, v_cache, licp)
    ko, vo = _cache_update_tc(xk, xv, k_cache, v_cache, li, cp)
    return keys, values, ko, vo
